# Initial kernel scaffold; baseline (speedup 1.0000x reference)
import jax
import jax.numpy as jnp
from jax.experimental import pallas as pl


def kernel(x_base, x_b62, x_esm, edge_dist, edge_is_seq, edge_inv_dist, params, edge_index, edge_seqbin, batch):
    raise NotImplementedError("write your pallas kernel here")


# speedup vs baseline: 2.5010x; 2.5010x over previous
import functools

import jax
import jax.numpy as jnp
from jax import lax
from jax.experimental import pallas as pl
from jax.experimental.pallas import tpu as pltpu
from jax.experimental.pallas import tpu_sc as plsc

N = 10000
G = 64
H = 256
DH = 128
E = 160000
NSUB = 16
CHUNK = 128
NCHUNK = 79
TPT = NCHUNK * CHUNK
E_PAD = NSUB * TPT
ROWS_SH = 10240
ROW_STRIPE = ROWS_SH // NSUB
OUT_STRIPE = 632
OUT_LAST = N - (NSUB - 1) * OUT_STRIPE
BN = 1000
BE = 2048
NBE = E_PAD // BE
NLAYERS = 5

_SQRT1_2 = 0.7071067811865476


def _gelu(x):
    return 0.5 * x * (1.0 + lax.erf(x * _SQRT1_2))


def _ln(x, g, b):
    mu = jnp.mean(x, axis=-1, keepdims=True)
    xc = x - mu
    var = jnp.mean(xc * xc, axis=-1, keepdims=True)
    return xc * lax.rsqrt(var + 1e-5) * g + b


def _full_spec(a):
    nd = a.ndim
    return pl.BlockSpec(a.shape, lambda i, _nd=nd: (0,) * _nd)


def _encoder_body(xb_ref, x62_ref, xe_ref,
                  bg, bb, bw1, bb1, bw2, bb2,
                  cg, cb, cw1, cb1, cw2, cb2,
                  eg, eb, ew1, eb1, ew2, eb2,
                  stg, stb, stw, stbb,
                  gw1, gb1, gw2, gb2,
                  fg, fb, fw, fbb,
                  o0_ref, o1_ref):
    def mlp2(x, g, b, w1, b1, w2, b2):
        h = _ln(x, g[0, :], b[0, :])
        h = _gelu(jnp.dot(h, w1[...], preferred_element_type=jnp.float32) + b1[0, :])
        h = _gelu(jnp.dot(h, w2[...], preferred_element_type=jnp.float32) + b2[0, :])
        return h

    hb = mlp2(xb_ref[...], bg, bb, bw1, bb1, bw2, bb2)
    h62 = mlp2(x62_ref[...], cg, cb, cw1, cb1, cw2, cb2)
    he = mlp2(xe_ref[...], eg, eb, ew1, eb1, ew2, eb2)
    hs = jnp.concatenate([hb, h62], axis=1)
    hs = _gelu(jnp.dot(_ln(hs, stg[0, :], stb[0, :]), stw[...],
                       preferred_element_type=jnp.float32) + stbb[0, :])
    gin = jnp.concatenate([hs, he], axis=1)
    gate = jax.nn.sigmoid(
        jnp.dot(_gelu(jnp.dot(gin, gw1[...], preferred_element_type=jnp.float32) + gb1[0, :]),
                gw2[...], preferred_element_type=jnp.float32) + gb2[0, :])
    he = he * gate
    fin = jnp.concatenate([hs, he], axis=1)
    x = _gelu(jnp.dot(_ln(fin, fg[0, :], fb[0, :]), fw[...],
                      preferred_element_type=jnp.float32) + fbb[0, :])
    o0_ref[...] = x[:, :DH]
    o1_ref[...] = x[:, DH:]


def _encoder_call(xb, x62, xe, weights):
    in_specs = [
        pl.BlockSpec((BN, xb.shape[1]), lambda i: (i, 0)),
        pl.BlockSpec((BN, x62.shape[1]), lambda i: (i, 0)),
        pl.BlockSpec((BN, xe.shape[1]), lambda i: (i, 0)),
    ] + [_full_spec(w) for w in weights]
    return pl.pallas_call(
        _encoder_body,
        grid=(N // BN,),
        in_specs=in_specs,
        out_specs=[pl.BlockSpec((BN, DH), lambda i: (i, 0))] * 2,
        out_shape=[jax.ShapeDtypeStruct((N, DH), jnp.float32)] * 2,
    )(xb, x62, xe, *weights)


def _edge_body(d_ref, is_ref, iv_ref, sb_ref, semb, w1r, w1s, w1i, w1v, b1,
               w2, b2, o0_ref, o1_ref):
    d = d_ref[0, 0, :]
    centers = lax.broadcasted_iota(jnp.int32, (BE, 32), 1).astype(jnp.float32) * (20.0 / 31.0)
    t = (d[:, None] - centers) * (31.0 / 20.0)
    rbf = jnp.exp(-(t * t))
    sbv = sb_ref[0, 0, :]
    oh = (sbv[:, None] == lax.broadcasted_iota(jnp.int32, (BE, 9), 1)).astype(jnp.float32)
    sw = jnp.dot(semb[...], w1s[...], preferred_element_type=jnp.float32)
    pre = (jnp.dot(rbf, w1r[...], preferred_element_type=jnp.float32)
           + jnp.dot(oh, sw, preferred_element_type=jnp.float32)
           + is_ref[0, 0, :][:, None] * w1i[0, :]
           + iv_ref[0, 0, :][:, None] * w1v[0, :]
           + b1[0, :])
    h = jnp.dot(_gelu(pre), w2[...], preferred_element_type=jnp.float32) + b2[0, :]
    o0_ref[...] = h[:, :DH]
    o1_ref[...] = h[:, DH:]


def _edge_call(d3, is3, iv3, sb3, semb, w1r, w1s, w1i, w1v, b1, w2, b2):
    e_spec = pl.BlockSpec((1, 1, BE), lambda i: (i, 0, 0))
    in_specs = [e_spec, e_spec, e_spec, e_spec] + [
        _full_spec(a) for a in (semb, w1r, w1s, w1i, w1v, b1, w2, b2)]
    return pl.pallas_call(
        _edge_body,
        grid=(NBE,),
        in_specs=in_specs,
        out_specs=[pl.BlockSpec((BE, DH), lambda i: (i, 0))] * 2,
        out_shape=[jax.ShapeDtypeStruct((E_PAD, DH), jnp.float32)] * 2,
    )(d3, is3, iv3, sb3, semb, w1r, w1s, w1i, w1v, b1, w2, b2)


def _make_mp_kernel():
    mesh = plsc.VectorSubcoreMesh(core_axis_name="c", subcore_axis_name="s")

    @functools.partial(
        pl.kernel,
        mesh=mesh,
        out_type=[jax.ShapeDtypeStruct((N, DH), jnp.float32),
                  jax.ShapeDtypeStruct((N, DH), jnp.float32)],
        scratch_types=[
            pltpu.VMEM((CHUNK,), jnp.int32),
            pltpu.VMEM((CHUNK,), jnp.int32),
            pltpu.VMEM((CHUNK, DH), jnp.float32),
            pltpu.VMEM((CHUNK, DH), jnp.float32),
            pltpu.VMEM_SHARED((ROWS_SH, DH), jnp.float32),
            pltpu.SemaphoreType.DMA,
        ],
    )
    def mp(x0, x1, ea0, ea1, src, dst, out0, out1,
           src_v, dst_v, xg_v, ea_v, acc_sh, sem):
        c = lax.axis_index("c")
        s = lax.axis_index("s")

        zero = jnp.zeros((16,), jnp.float32)

        def zrow(r, carry):
            for j in range(DH // 16):
                xg_v[r, pl.ds(j * 16, 16)] = zero
            return carry

        lax.fori_loop(0, CHUNK, zrow, 0)
        for i in range(ROW_STRIPE // CHUNK):
            pltpu.sync_copy(xg_v, acc_sh.at[pl.ds(s * ROW_STRIPE + i * CHUNK, CHUNK)])
        plsc.subcore_barrier()

        def accum(x_hbm, ea_hbm):
            def body(k, carry):
                off = s * TPT + k * CHUNK
                pltpu.sync_copy(src.at[pl.ds(off, CHUNK)], src_v)
                pltpu.sync_copy(dst.at[pl.ds(off, CHUNK)], dst_v)
                pltpu.async_copy(x_hbm.at[src_v], xg_v, sem).wait()
                pltpu.sync_copy(ea_hbm.at[pl.ds(off, CHUNK)], ea_v)

                def row(r, rc):
                    for j in range(DH // 16):
                        sl = pl.ds(j * 16, 16)
                        ea_v[r, sl] = jnp.maximum(xg_v[r, sl] + ea_v[r, sl], 0.0)
                    return rc

                lax.fori_loop(0, CHUNK, row, 0)
                pltpu.sync_copy(ea_v, acc_sh.at[dst_v], add=True)
                return carry

            lax.fori_loop(0, NCHUNK, body, 0)

        @pl.when(c == 0)
        def _():
            accum(x0, ea0)

        @pl.when(c == 1)
        def _():
            accum(x1, ea1)

        plsc.subcore_barrier()

        def writeout(out_hbm):
            @pl.when(s < NSUB - 1)
            def _():
                pltpu.sync_copy(acc_sh.at[pl.ds(s * OUT_STRIPE, OUT_STRIPE)],
                                out_hbm.at[pl.ds(s * OUT_STRIPE, OUT_STRIPE)])

            @pl.when(s == NSUB - 1)
            def _():
                pltpu.sync_copy(acc_sh.at[pl.ds((NSUB - 1) * OUT_STRIPE, OUT_LAST)],
                                out_hbm.at[pl.ds((NSUB - 1) * OUT_STRIPE, OUT_LAST)])

        @pl.when(c == 0)
        def _():
            writeout(out0)

        @pl.when(c == 1)
        def _():
            writeout(out1)

    return mp


def _conv_body(x0_ref, x1_ref, a0_ref, a1_ref, w1, b1, w2, b2, g, bn,
               o0_ref, o1_ref):
    xin = jnp.concatenate([x0_ref[...], x1_ref[...]], axis=1)
    sv = xin + jnp.concatenate([a0_ref[...], a1_ref[...]], axis=1)
    h = _gelu(jnp.dot(sv, w1[...], preferred_element_type=jnp.float32) + b1[0, :])
    h = jnp.dot(h, w2[...], preferred_element_type=jnp.float32) + b2[0, :]
    h = _gelu(_ln(h, g[0, :], bn[0, :]))
    xn = h + xin
    o0_ref[...] = xn[:, :DH]
    o1_ref[...] = xn[:, DH:]


def _conv_call(x0, x1, a0, a1, w1, b1, w2, b2, g, bn):
    n_spec = pl.BlockSpec((BN, DH), lambda i: (i, 0))
    in_specs = [n_spec] * 4 + [_full_spec(a) for a in (w1, b1, w2, b2, g, bn)]
    return pl.pallas_call(
        _conv_body,
        grid=(N // BN,),
        in_specs=in_specs,
        out_specs=[n_spec] * 2,
        out_shape=[jax.ShapeDtypeStruct((N, DH), jnp.float32)] * 2,
    )(x0, x1, a0, a1, w1, b1, w2, b2, g, bn)


def _readout_body(x0_ref, x1_ref, brow_ref, bcol_ref, rw, rb, o_ref):
    brow = brow_ref[...]
    gids = lax.broadcasted_iota(jnp.int32, (G, N), 0)
    oh = (gids == brow).astype(jnp.float32)
    counts = jnp.sum(oh, axis=1)
    x0 = x0_ref[...]
    x1 = x1_ref[...]
    inv = 1.0 / jnp.maximum(counts, 1.0)
    m0 = jnp.dot(oh, x0, preferred_element_type=jnp.float32) * inv[:, None]
    m1 = jnp.dot(oh, x1, preferred_element_type=jnp.float32) * inv[:, None]
    bcol = bcol_ref[...]
    giota = lax.broadcasted_iota(jnp.int32, (G, 1), 0)

    def gbody(gi, mxacc):
        mask = bcol == gi
        mm0 = jnp.max(jnp.where(mask, x0, -3.4e38), axis=0)
        mm1 = jnp.max(jnp.where(mask, x1, -3.4e38), axis=0)
        row = jnp.concatenate([mm0, mm1])[None, :]
        return jnp.where(giota == gi, row, mxacc)

    mx = lax.fori_loop(0, G, gbody, jnp.full((G, H), -3.4e38, jnp.float32))
    mx = jnp.where(counts[:, None] > 0.0, mx, 0.0)
    feat = jnp.concatenate([m0, m1, mx], axis=1)
    o_ref[...] = _gelu(jnp.dot(feat, rw[...], preferred_element_type=jnp.float32) + rb[0, :])


def _readout_call(x0, x1, brow, bcol, rw, rb):
    in_specs = [_full_spec(a) for a in (x0, x1, brow, bcol, rw, rb)]
    return pl.pallas_call(
        _readout_body,
        grid=(1,),
        in_specs=in_specs,
        out_specs=pl.BlockSpec((G, H), lambda i: (0, 0)),
        out_shape=jax.ShapeDtypeStruct((G, H), jnp.float32),
    )(x0, x1, brow, bcol, rw, rb)


def kernel(x_base, x_b62, x_esm, edge_dist, edge_is_seq, edge_inv_dist,
           params, edge_index, edge_seqbin, batch):
    p = params

    def v2(a):
        return a.reshape(1, -1).astype(jnp.float32)

    enc_w = (
        v2(p["base"]["ln_g"]), v2(p["base"]["ln_b"]),
        p["base"]["l1"]["w"], v2(p["base"]["l1"]["b"]),
        p["base"]["l2"]["w"], v2(p["base"]["l2"]["b"]),
        v2(p["b62"]["ln_g"]), v2(p["b62"]["ln_b"]),
        p["b62"]["l1"]["w"], v2(p["b62"]["l1"]["b"]),
        p["b62"]["l2"]["w"], v2(p["b62"]["l2"]["b"]),
        v2(p["esm"]["ln_g"]), v2(p["esm"]["ln_b"]),
        p["esm"]["l1"]["w"], v2(p["esm"]["l1"]["b"]),
        p["esm"]["l2"]["w"], v2(p["esm"]["l2"]["b"]),
        v2(p["struct"]["ln_g"]), v2(p["struct"]["ln_b"]),
        p["struct"]["l"]["w"], v2(p["struct"]["l"]["b"]),
        p["gate"]["l1"]["w"], v2(p["gate"]["l1"]["b"]),
        p["gate"]["l2"]["w"], v2(p["gate"]["l2"]["b"]),
        v2(p["fuse"]["ln_g"]), v2(p["fuse"]["ln_b"]),
        p["fuse"]["l"]["w"], v2(p["fuse"]["l"]["b"]),
    )
    x0, x1 = _encoder_call(x_base, x_b62, x_esm, enc_w)

    pad = E_PAD - E
    ew1 = p["edge"]["l1"]["w"]
    d3 = jnp.pad(edge_dist, (0, pad)).reshape(NBE, 1, BE)
    is3 = jnp.pad(edge_is_seq, (0, pad)).reshape(NBE, 1, BE)
    iv3 = jnp.pad(edge_inv_dist, (0, pad)).reshape(NBE, 1, BE)
    sb3 = jnp.pad(edge_seqbin.astype(jnp.int32), (0, pad)).reshape(NBE, 1, BE)
    ea0, ea1 = _edge_call(
        d3, is3, iv3, sb3, p["seq_emb"],
        ew1[0:32], ew1[32:48], ew1[48:49], ew1[49:50],
        v2(p["edge"]["l1"]["b"]), p["edge"]["l2"]["w"], v2(p["edge"]["l2"]["b"]))

    src = jnp.pad(edge_index[0].astype(jnp.int32), (0, pad))
    dst = jnp.pad(edge_index[1].astype(jnp.int32), (0, pad), constant_values=N)

    mp = _make_mp_kernel()
    for i in range(NLAYERS):
        a0, a1 = mp(x0, x1, ea0, ea1, src, dst)
        cp = p["convs"][i]
        nm = p["norms"][i]
        x0, x1 = _conv_call(x0, x1, a0, a1,
                            cp["l1"]["w"], v2(cp["l1"]["b"]),
                            cp["l2"]["w"], v2(cp["l2"]["b"]),
                            v2(nm["g"]), v2(nm["b"]))

    bi = batch.astype(jnp.int32)
    return _readout_call(x0, x1, bi.reshape(1, N), bi.reshape(N, 1),
                         p["readout"]["l"]["w"], v2(p["readout"]["l"]["b"]))
